# Initial kernel scaffold; baseline (speedup 1.0000x reference)
import functools

import jax
import jax.numpy as jnp
from jax import lax
from jax.experimental import pallas as pl
from jax.experimental.pallas import tpu as pltpu
from jax.experimental.pallas import tpu_sc as plsc

B, L, V, D = 16384, 200, 81, 64
T = B * L
NC, NS = 2, 16
NW = NC * NS
W = T // NW
IPD = 128
KD = 8
S = IPD * KD
STEPS = W // S
XROWS = T // IPD


def _body(table_hbm, idx_hbm, out_hbm, idx_v, rows_v, sem):
    wid = lax.axis_index("s") * NC + lax.axis_index("c")
    row_base = wid * (W // IPD)
    out_base = wid * W

    def step(ib, carry):
        pltpu.sync_copy(idx_hbm.at[pl.ds(row_base + ib * KD, KD)], idx_v)
        copies = [
            pltpu.async_copy(
                table_hbm.at[idx_v.at[j]],
                rows_v.at[pl.ds(j * IPD, IPD)],
                sem,
            )
            for j in range(KD)
        ]
        for c in copies:
            c.wait()
        pltpu.sync_copy(rows_v, out_hbm.at[pl.ds(out_base + ib * S, S)])
        return carry

    lax.fori_loop(0, STEPS, step, 0)


_emb = functools.partial(
    pl.kernel,
    out_type=jax.ShapeDtypeStruct((T, D), jnp.float32),
    mesh=plsc.VectorSubcoreMesh(
        core_axis_name="c", subcore_axis_name="s", num_cores=NC, num_subcores=NS
    ),
    scratch_types=[
        pltpu.VMEM((KD, IPD), jnp.int32),
        pltpu.VMEM((S, D), jnp.float32),
        pltpu.SemaphoreType.DMA,
    ],
)(_body)


def kernel(x, table):
    out = _emb(table, x.reshape(XROWS, IPD))
    return out.reshape(B, L * D)


# speedup vs baseline: 3.8730x; 3.8730x over previous
import functools

import jax
import jax.numpy as jnp
from jax import lax
from jax.experimental import pallas as pl
from jax.experimental.pallas import tpu as pltpu
from jax.experimental.pallas import tpu_sc as plsc

B, L, V, D = 16384, 200, 81, 64
T = B * L
NC, NS = 2, 16
NW = NC * NS
W = T // NW
IPD = 128
KD = 8
S = IPD * KD
STEPS = W // S
XROWS = T // IPD


def _body(table_hbm, idx_hbm, out_hbm, idx_v, rows_v, sem):
    wid = lax.axis_index("s") * NC + lax.axis_index("c")
    row_base = wid * (W // IPD)
    out_base = wid * W

    def step(ib, carry):
        pltpu.sync_copy(idx_hbm.at[pl.ds(row_base + ib * KD, KD)], idx_v)
        copies = [
            pltpu.async_copy(
                table_hbm.at[idx_v.at[j]],
                rows_v.at[pl.ds(j * IPD, IPD)],
                sem,
            )
            for j in range(KD)
        ]
        for c in copies:
            c.wait()
        pltpu.sync_copy(rows_v, out_hbm.at[pl.ds(out_base + ib * S, S)])
        return carry

    lax.fori_loop(0, STEPS, step, 0)


_emb = functools.partial(
    pl.kernel,
    out_type=jax.ShapeDtypeStruct((T, D), jnp.float32),
    mesh=plsc.VectorSubcoreMesh(
        core_axis_name="c", subcore_axis_name="s", num_cores=NC, num_subcores=NS
    ),
    scratch_types=[
        pltpu.VMEM((KD, IPD), jnp.int32),
        pltpu.VMEM((S, D), jnp.float32),
        pltpu.SemaphoreType.DMA,
    ],
    compiler_params=pltpu.CompilerParams(use_tc_tiling_on_sc=False),
)(_body)


def kernel(x, table):
    out = _emb(table, x.reshape(XROWS, IPD))
    return out.reshape(B, L * D)


# speedup vs baseline: 9.8846x; 2.5522x over previous
import functools

import jax
import jax.numpy as jnp
from jax import lax
from jax.experimental import pallas as pl
from jax.experimental.pallas import tpu as pltpu
from jax.experimental.pallas import tpu_sc as plsc

B, L, V, D = 16384, 200, 81, 64
T = B * L
NC, NS = 2, 16
NW = NC * NS
W = T // NW
IPD = 128
KD = 8
S = IPD * KD
STEPS = W // S
XROWS = T // IPD


def _body(table_hbm, idx_hbm, out_hbm, table_v, idx_v, rows_v, sem):
    wid = lax.axis_index("s") * NC + lax.axis_index("c")
    row_base = wid * (W // IPD)
    out_base = wid * W

    @pl.when(lax.axis_index("s") == 0)
    def _():
        pltpu.sync_copy(table_hbm, table_v)

    plsc.subcore_barrier()

    def step(ib, carry):
        pltpu.sync_copy(idx_hbm.at[pl.ds(row_base + ib * KD, KD)], idx_v)
        copies = [
            pltpu.async_copy(
                table_v.at[idx_v.at[j]],
                rows_v.at[pl.ds(j * IPD, IPD)],
                sem,
            )
            for j in range(KD)
        ]
        for c in copies:
            c.wait()
        pltpu.sync_copy(rows_v, out_hbm.at[pl.ds(out_base + ib * S, S)])
        return carry

    lax.fori_loop(0, STEPS, step, 0)


_emb = functools.partial(
    pl.kernel,
    out_type=jax.ShapeDtypeStruct((T, D), jnp.float32),
    mesh=plsc.VectorSubcoreMesh(
        core_axis_name="c", subcore_axis_name="s", num_cores=NC, num_subcores=NS
    ),
    scratch_types=[
        pltpu.VMEM_SHARED((V, D), jnp.float32),
        pltpu.VMEM((KD, IPD), jnp.int32),
        pltpu.VMEM((S, D), jnp.float32),
        pltpu.SemaphoreType.DMA,
    ],
    compiler_params=pltpu.CompilerParams(use_tc_tiling_on_sc=False),
)(_body)


def kernel(x, table):
    out = _emb(table, x.reshape(XROWS, IPD))
    return out.reshape(B, L * D)


# speedup vs baseline: 33.0193x; 3.3405x over previous
import functools

import jax
import jax.numpy as jnp
from jax import lax
from jax.experimental import pallas as pl
from jax.experimental.pallas import tpu as pltpu
from jax.experimental.pallas import tpu_sc as plsc

B, L, V, D = 16384, 200, 81, 64
VP = 88
P = V * VP
T = B * L
NC, NS = 2, 16
NW = NC * NS
BR = B // NW
GR = 128
NG = BR // GR
PC = L // 2
XR = L * GR // 128
XROWS2 = T // 128


def _body(table_hbm, idx_hbm, out_hbm, pair_s, xs_v, pidx_v, wb0, wb1,
          gsem0, gsem1, wsem0, wsem1):
    cid = lax.axis_index("c")
    sid = lax.axis_index("s")
    wid = sid * NC + cid
    xrow_base = wid * (L * BR // 128)
    orow_base = wid * BR

    pltpu.sync_copy(table_hbm, wb1.at[pl.ds(0, VP)])

    def build_i(k, carry):
        i = sid + k * NS

        @pl.when(i < V)
        def _():
            lefts = [wb1[i, pl.ds(c * 16, 16)] for c in range(D // 16)]

            def put(j, carry2):
                for c in range(D // 16):
                    wb0[j, pl.ds(c * 16, 16)] = lefts[c]
                    wb0[j, pl.ds(D + c * 16, 16)] = wb1[j, pl.ds(c * 16, 16)]
                return carry2

            lax.fori_loop(0, V, put, 0)
            pltpu.sync_copy(wb0.at[pl.ds(0, VP)], pair_s.at[pl.ds(i * VP, VP)])
        return carry

    lax.fori_loop(0, (V + NS - 1) // NS, build_i, 0)
    plsc.subcore_barrier()

    wbufs = (wb0, wb1)
    gsems = (gsem0, gsem1)
    iot = lax.iota(jnp.int32, 16)

    def compute_pidx(c, carry):
        for q in range(GR // 16):
            flat_e = (q * 16 + iot) * L + 2 * c
            flat_o = flat_e + 1
            even = plsc.load_gather(
                xs_v, [lax.shift_right_logical(flat_e, 7), flat_e & 127]
            )
            odd = plsc.load_gather(
                xs_v, [lax.shift_right_logical(flat_o, 7), flat_o & 127]
            )
            pidx_v[c, pl.ds(q * 16, 16)] = even * VP + odd
        return carry

    def fire(blk, buf):
        return pltpu.async_copy(
            pair_s.at[pidx_v.at[blk]], wbufs[buf], gsems[buf]
        )

    def group(g, carry):
        pltpu.sync_copy(idx_hbm.at[pl.ds(xrow_base + g * XR, XR)], xs_v)
        lax.fori_loop(0, PC, compute_pidx, 0)
        r0 = orow_base + g * GR

        def out_slice(blk):
            return out_hbm.at[pl.ds(r0, GR), pl.ds(blk * 128, 128)]

        def block2(h, carry2):
            b0 = 2 * h
            c0 = fire(b0, 0)

            @pl.when(h > 0)
            def _():
                pltpu.async_copy(wb1, out_slice(b0 - 1), wsem1)

            c0.wait()

            @pl.when(h > 0)
            def _():
                pltpu.make_async_copy(wb1, out_slice(b0 - 1), wsem1).wait()

            c1 = fire(b0 + 1, 1)
            w0 = pltpu.async_copy(wb0, out_slice(b0), wsem0)
            c1.wait()
            w0.wait()
            return carry2

        lax.fori_loop(0, PC // 2, block2, 0)
        pltpu.async_copy(wb1, out_slice(PC - 1), wsem1).wait()
        return carry

    lax.fori_loop(0, NG, group, 0)


_emb = functools.partial(
    pl.kernel,
    out_type=jax.ShapeDtypeStruct((B, L * D), jnp.float32),
    mesh=plsc.VectorSubcoreMesh(
        core_axis_name="c", subcore_axis_name="s", num_cores=NC, num_subcores=NS
    ),
    scratch_types=[
        pltpu.VMEM_SHARED((P, 128), jnp.float32),
        pltpu.VMEM((XR, 128), jnp.int32),
        pltpu.VMEM((PC, GR), jnp.int32),
        pltpu.VMEM((GR, 128), jnp.float32),
        pltpu.VMEM((GR, 128), jnp.float32),
        pltpu.SemaphoreType.DMA,
        pltpu.SemaphoreType.DMA,
        pltpu.SemaphoreType.DMA,
        pltpu.SemaphoreType.DMA,
    ],
    compiler_params=pltpu.CompilerParams(needs_layout_passes=False),
)(_body)


def kernel(x, table):
    table_p = jnp.pad(table, ((0, VP - V), (0, 128 - D)))
    return _emb(table_p, x.reshape(XROWS2, 128))


# speedup vs baseline: 34.3029x; 1.0389x over previous
import functools

import jax
import jax.numpy as jnp
from jax import lax
from jax.experimental import pallas as pl
from jax.experimental.pallas import tpu as pltpu
from jax.experimental.pallas import tpu_sc as plsc

B, L, V, D = 16384, 200, 81, 64
VP = 88
P = V * VP
T = B * L
NC, NS = 2, 16
NW = NC * NS
BR = B // NW
GR = 128
NG = BR // GR
PC = L // 2
XR = L * GR // 128
XROWS2 = T // 128


def _body(table_hbm, idx_hbm, out_hbm, pair_s, xs_v, pidx_v, wb0, wb1,
          gsem0, gsem1, wsem0, wsem1):
    cid = lax.axis_index("c")
    sid = lax.axis_index("s")
    wid = sid * NC + cid
    xrow_base = wid * (L * BR // 128)
    orow_base = wid * BR

    pltpu.sync_copy(table_hbm, wb1.at[pl.ds(0, VP)])

    def build_i(k, carry):
        i = sid + k * NS

        @pl.when(i < V)
        def _():
            lefts = [wb1[i, pl.ds(c * 16, 16)] for c in range(D // 16)]

            def put(j, carry2):
                for c in range(D // 16):
                    wb0[j, pl.ds(c * 16, 16)] = lefts[c]
                    wb0[j, pl.ds(D + c * 16, 16)] = wb1[j, pl.ds(c * 16, 16)]
                return carry2

            lax.fori_loop(0, V, put, 0)
            pltpu.sync_copy(wb0.at[pl.ds(0, VP)], pair_s.at[pl.ds(i * VP, VP)])
        return carry

    lax.fori_loop(0, (V + NS - 1) // NS, build_i, 0)
    plsc.subcore_barrier()

    wbufs = (wb0, wb1)
    gsems = (gsem0, gsem1)
    iot = lax.iota(jnp.int32, 16)

    def compute_pidx(c, carry):
        for q in range(GR // 16):
            flat_e = (q * 16 + iot) * L + 2 * c
            flat_o = flat_e + 1
            even = plsc.load_gather(
                xs_v, [lax.shift_right_logical(flat_e, 7), flat_e & 127]
            )
            odd = plsc.load_gather(
                xs_v, [lax.shift_right_logical(flat_o, 7), flat_o & 127]
            )
            pidx_v[c, pl.ds(q * 16, 16)] = even * VP + odd
        return carry

    def fire(blk, buf):
        return pltpu.async_copy(
            pair_s.at[pidx_v.at[blk]], wbufs[buf], gsems[buf]
        )

    def group(g, carry):
        pltpu.sync_copy(idx_hbm.at[pl.ds(xrow_base + g * XR, XR)], xs_v)
        compute_pidx(0, 0)
        compute_pidx(1, 0)
        r0 = orow_base + g * GR

        def out_slice(blk):
            return out_hbm.at[pl.ds(r0, GR), pl.ds(blk * 128, 128)]

        def block2(h, carry2):
            b0 = 2 * h
            c0 = fire(b0, 0)

            @pl.when(h > 0)
            def _():
                pltpu.async_copy(wb1, out_slice(b0 - 1), wsem1)

            @pl.when(b0 + 2 < PC)
            def _():
                compute_pidx(b0 + 2, 0)

            c0.wait()

            @pl.when(h > 0)
            def _():
                pltpu.make_async_copy(wb1, out_slice(b0 - 1), wsem1).wait()

            c1 = fire(b0 + 1, 1)
            w0 = pltpu.async_copy(wb0, out_slice(b0), wsem0)

            @pl.when(b0 + 3 < PC)
            def _():
                compute_pidx(b0 + 3, 0)

            c1.wait()
            w0.wait()
            return carry2

        lax.fori_loop(0, PC // 2, block2, 0)
        pltpu.async_copy(wb1, out_slice(PC - 1), wsem1).wait()
        return carry

    lax.fori_loop(0, NG, group, 0)


_emb = functools.partial(
    pl.kernel,
    out_type=jax.ShapeDtypeStruct((B, L * D), jnp.float32),
    mesh=plsc.VectorSubcoreMesh(
        core_axis_name="c", subcore_axis_name="s", num_cores=NC, num_subcores=NS
    ),
    scratch_types=[
        pltpu.VMEM_SHARED((P, 128), jnp.float32),
        pltpu.VMEM((XR, 128), jnp.int32),
        pltpu.VMEM((PC, GR), jnp.int32),
        pltpu.VMEM((GR, 128), jnp.float32),
        pltpu.VMEM((GR, 128), jnp.float32),
        pltpu.SemaphoreType.DMA,
        pltpu.SemaphoreType.DMA,
        pltpu.SemaphoreType.DMA,
        pltpu.SemaphoreType.DMA,
    ],
    compiler_params=pltpu.CompilerParams(needs_layout_passes=False),
)(_body)


def kernel(x, table):
    table_p = jnp.pad(table, ((0, VP - V), (0, 128 - D)))
    return _emb(table_p, x.reshape(XROWS2, 128))
